# baseline (speedup 1.0000x reference)
import jax
import jax.numpy as jnp
from jax.experimental import pallas as pl


def _rq_body(x_ref, cm2_ref, packed_ref, b2_ref, recon_ref, codes_ref):
    x = x_ref[...]
    L, K, D = cm2_ref.shape
    bm = x.shape[0]
    iota = jax.lax.broadcasted_iota(jnp.int32, (bm, K), 1)
    ones = jnp.ones((D, 1), dtype=jnp.float32)

    r = x
    recon = jnp.zeros_like(x)
    for l in range(L):
        a2 = jax.lax.dot_general(
            r * r, ones, (((1,), (0,)), ((), ())),
            precision=jax.lax.Precision.HIGHEST,
            preferred_element_type=jnp.float32)
        b2 = b2_ref[l]
        ab2 = jax.lax.dot_general(
            r, cm2_ref[l], (((1,), (1,)), ((), ())),
            preferred_element_type=jnp.float32)
        d2 = jnp.maximum(a2 + b2 + ab2, 0.0)
        m = jnp.min(d2, axis=1, keepdims=True)
        idx = jnp.min(jnp.where(d2 <= m, iota, K), axis=1,
                      keepdims=True)
        codes_ref[:, l:l + 1] = idx
        onehot = (iota == idx).astype(jnp.bfloat16)
        parts = jax.lax.dot_general(
            onehot, packed_ref[l], (((1,), (0,)), ((), ())),
            preferred_element_type=jnp.float32)
        sel = (parts[:, :D] + parts[:, D:2 * D]) + parts[:, 2 * D:]
        recon = recon + sel
        r = r - sel
    recon_ref[...] = recon


def kernel(x, codebooks):
    B, D = x.shape
    L, K, _ = codebooks.shape
    bm = 512

    cm2 = -2.0 * codebooks
    c1 = codebooks.astype(jnp.bfloat16)
    r1 = codebooks - c1.astype(jnp.float32)
    c2 = r1.astype(jnp.bfloat16)
    c3 = (r1 - c2.astype(jnp.float32)).astype(jnp.bfloat16)
    packed = jnp.concatenate([c1, c2, c3], axis=2)
    b2 = jnp.sum(codebooks * codebooks, axis=2)[:, None, :]

    recon, codes_pad = pl.pallas_call(
        _rq_body,
        grid=(B // bm,),
        in_specs=[
            pl.BlockSpec((bm, D), lambda i: (i, 0)),
            pl.BlockSpec((L, K, D), lambda i: (0, 0, 0)),
            pl.BlockSpec((L, K, 3 * D), lambda i: (0, 0, 0)),
            pl.BlockSpec((L, 1, K), lambda i: (0, 0, 0)),
        ],
        out_specs=[
            pl.BlockSpec((bm, D), lambda i: (i, 0)),
            pl.BlockSpec((bm, 8), lambda i: (i, 0)),
        ],
        out_shape=[
            jax.ShapeDtypeStruct((B, D), jnp.float32),
            jax.ShapeDtypeStruct((B, 8), jnp.int32),
        ],
    )(x, cm2, packed, b2)
    return recon, codes_pad[:, :L]


# speedup vs baseline: 1.1156x; 1.1156x over previous
import jax
import jax.numpy as jnp
from jax.experimental import pallas as pl


def _rq_body(x_ref, cm2_ref, packed_ref, b2_ref, recon_ref, codes_ref):
    x = x_ref[...]
    L, K, D = cm2_ref.shape
    bm = x.shape[0]
    iota = jax.lax.broadcasted_iota(jnp.int32, (bm, K), 1)
    ones = jnp.ones((D, 1), dtype=jnp.float32)

    r = x
    recon = jnp.zeros_like(x)
    for l in range(L):
        a2 = jax.lax.dot_general(
            r * r, ones, (((1,), (0,)), ((), ())),
            precision=jax.lax.Precision.HIGHEST,
            preferred_element_type=jnp.float32)
        b2 = b2_ref[l]
        ab2 = jax.lax.dot_general(
            r, cm2_ref[l], (((1,), (1,)), ((), ())),
            preferred_element_type=jnp.float32)
        d2 = jnp.maximum(a2 + b2 + ab2, 0.0)
        m = jnp.min(d2, axis=1, keepdims=True)
        idx = jnp.min(jnp.where(d2 <= m, iota, K), axis=1,
                      keepdims=True)
        codes_ref[:, l:l + 1] = idx
        onehot = (iota == idx).astype(jnp.bfloat16)
        parts = jax.lax.dot_general(
            onehot, packed_ref[l], (((1,), (0,)), ((), ())),
            preferred_element_type=jnp.float32)
        sel = (parts[:, :D] + parts[:, D:2 * D]) + parts[:, 2 * D:]
        recon = recon + sel
        r = r - sel
    recon_ref[...] = recon


def kernel(x, codebooks):
    B, D = x.shape
    L, K, _ = codebooks.shape
    bm = 1024

    cm2 = -2.0 * codebooks
    c1 = codebooks.astype(jnp.bfloat16)
    r1 = codebooks - c1.astype(jnp.float32)
    c2 = r1.astype(jnp.bfloat16)
    c3 = (r1 - c2.astype(jnp.float32)).astype(jnp.bfloat16)
    packed = jnp.concatenate([c1, c2, c3], axis=2)
    b2 = jnp.sum(codebooks * codebooks, axis=2)[:, None, :]

    recon, codes_pad = pl.pallas_call(
        _rq_body,
        grid=(B // bm,),
        in_specs=[
            pl.BlockSpec((bm, D), lambda i: (i, 0)),
            pl.BlockSpec((L, K, D), lambda i: (0, 0, 0)),
            pl.BlockSpec((L, K, 3 * D), lambda i: (0, 0, 0)),
            pl.BlockSpec((L, 1, K), lambda i: (0, 0, 0)),
        ],
        out_specs=[
            pl.BlockSpec((bm, D), lambda i: (i, 0)),
            pl.BlockSpec((bm, 8), lambda i: (i, 0)),
        ],
        out_shape=[
            jax.ShapeDtypeStruct((B, D), jnp.float32),
            jax.ShapeDtypeStruct((B, 8), jnp.int32),
        ],
    )(x, cm2, packed, b2)
    return recon, codes_pad[:, :L]


# speedup vs baseline: 1.2221x; 1.0955x over previous
import jax
import jax.numpy as jnp
from jax.experimental import pallas as pl


def _rq_body(x_ref, cm2_ref, packed_ref, b2_ref, recon_ref, codes_ref):
    x = x_ref[...]
    L, K, D = cm2_ref.shape
    bm = x.shape[0]
    iota = jax.lax.broadcasted_iota(jnp.int32, (bm, K), 1)

    r = x
    recon = jnp.zeros_like(x)
    for l in range(L):
        a2 = jnp.sum(r * r, axis=1, keepdims=True)
        b2 = b2_ref[l]
        ab2 = jax.lax.dot_general(
            r, cm2_ref[l], (((1,), (1,)), ((), ())),
            preferred_element_type=jnp.float32)
        d2 = jnp.maximum(a2 + b2 + ab2, 0.0)
        m = jnp.min(d2, axis=1, keepdims=True)
        idx = jnp.min(jnp.where(d2 <= m, iota, K), axis=1,
                      keepdims=True)
        codes_ref[:, l:l + 1] = idx
        onehot = (iota == idx).astype(jnp.bfloat16)
        parts = jax.lax.dot_general(
            onehot, packed_ref[l], (((1,), (0,)), ((), ())),
            preferred_element_type=jnp.float32)
        sel = (parts[:, :D] + parts[:, D:2 * D]) + parts[:, 2 * D:]
        recon = recon + sel
        r = r - sel
    recon_ref[...] = recon


def kernel(x, codebooks):
    B, D = x.shape
    L, K, _ = codebooks.shape
    bm = 1024

    cm2 = -2.0 * codebooks
    c1 = codebooks.astype(jnp.bfloat16)
    r1 = codebooks - c1.astype(jnp.float32)
    c2 = r1.astype(jnp.bfloat16)
    c3 = (r1 - c2.astype(jnp.float32)).astype(jnp.bfloat16)
    packed = jnp.concatenate([c1, c2, c3], axis=2)
    b2 = jnp.sum(codebooks * codebooks, axis=2)[:, None, :]

    recon, codes_pad = pl.pallas_call(
        _rq_body,
        grid=(B // bm,),
        in_specs=[
            pl.BlockSpec((bm, D), lambda i: (i, 0)),
            pl.BlockSpec((L, K, D), lambda i: (0, 0, 0)),
            pl.BlockSpec((L, K, 3 * D), lambda i: (0, 0, 0)),
            pl.BlockSpec((L, 1, K), lambda i: (0, 0, 0)),
        ],
        out_specs=[
            pl.BlockSpec((bm, D), lambda i: (i, 0)),
            pl.BlockSpec((bm, 8), lambda i: (i, 0)),
        ],
        out_shape=[
            jax.ShapeDtypeStruct((B, D), jnp.float32),
            jax.ShapeDtypeStruct((B, 8), jnp.int32),
        ],
    )(x, cm2, packed, b2)
    return recon, codes_pad[:, :L]
